# baseline (speedup 1.0000x reference)
import jax
import jax.numpy as jnp
import numpy as np
from jax.experimental import pallas as pl
from jax.experimental.pallas import tpu as pltpu

B = 1024
D = 32
M = 100000
K = 4096
BR = 16
NB = B // BR

W = 4096
NFULL = M // W
TAIL = M - NFULL * W

_FLIP32 = np.int32(0x7FFFFFFF)


def _sortable32(bits):
    return jnp.where(bits < 0, bits ^ _FLIP32, bits)


def _count_ge_bf(bf_ref, mid_bf):
    def chunk(c, acc):
        blk = bf_ref[:, pl.ds(c * W, W)]
        return acc + (blk >= mid_bf).astype(jnp.bfloat16)

    acc = jax.lax.fori_loop(0, NFULL, chunk, jnp.zeros((BR, W), jnp.bfloat16))
    cnt = jnp.sum(acc.astype(jnp.float32), axis=1, keepdims=True)
    tailm = bf_ref[:, pl.ds(NFULL * W, TAIL)] >= mid_bf
    return cnt + jnp.sum(tailm.astype(jnp.float32), axis=1, keepdims=True)


def _search_hi16(bf_ref, k, z):
    lo0 = jnp.full((BR, 1), -32768, jnp.int32)
    hi0 = jnp.full((BR, 1), 32767, jnp.int32)

    def body(_, carry):
        lo, hi = carry
        mid = (lo >> 1) + (hi >> 1) + ((lo | hi) & 1)
        mbits = jnp.where(mid < 0, mid ^ 0x7FFF, mid).astype(jnp.int16)
        mid_bf = jax.lax.bitcast_convert_type(mbits, jnp.bfloat16)
        cnt = _count_ge_bf(bf_ref, mid_bf) - jnp.where(mid == 0, z, 0.0)
        ge = cnt >= k
        return jnp.where(ge, mid, lo), jnp.where(ge, hi, mid - 1)

    lo, _ = jax.lax.fori_loop(0, 16, body, (lo0, hi0))
    return lo


def _search_byte(bf_ref, k):
    lo0 = jnp.full((BR, 1), 0, jnp.int32)
    hi0 = jnp.full((BR, 1), 255, jnp.int32)

    def body(_, carry):
        lo, hi = carry
        mid = (lo + hi + 1) >> 1
        cnt = _count_ge_bf(bf_ref, mid.astype(jnp.bfloat16))
        ge = cnt >= k
        return jnp.where(ge, mid, lo), jnp.where(ge, hi, mid - 1)

    lo, _ = jax.lax.fori_loop(0, 8, body, (lo0, hi0))
    return lo


def _tc_body(points_ref, idx_ref, mbT_ref, sims_ref, loss_ref, bf_ref, mb_vmem, sem):
    i = pl.program_id(0)

    @pl.when(i == 0)
    def _():
        cp = pltpu.make_async_copy(mbT_ref, mb_vmem, sem)
        cp.start()
        cp.wait()

    p = points_ref[...]
    norm = jnp.sqrt(jnp.sum(p * p, axis=1, keepdims=True))
    pn = p / norm
    sims = jnp.dot(pn, mb_vmem[...], preferred_element_type=jnp.float32)
    sims_ref[...] = sims

    h16 = (jax.lax.bitcast_convert_type(sims + 0.0, jnp.int32) >> 16).astype(
        jnp.int16
    )
    z = jnp.sum((h16 == jnp.int16(-32768)).astype(jnp.float32), axis=1, keepdims=True)
    bf_ref[...] = jax.lax.bitcast_convert_type(h16, jnp.bfloat16)
    kvec = jnp.full((BR, 1), float(K), jnp.float32)
    p_hi = _search_hi16(bf_ref, kvec, z)

    s32 = _sortable32(jax.lax.bitcast_convert_type(sims_ref[...] + 0.0, jnp.int32))
    hi = s32 >> 16
    tie1 = hi == p_hi
    c_hi = jnp.sum((hi > p_hi).astype(jnp.float32), axis=1, keepdims=True)
    b1 = (s32 >> 8) & 0xFF
    bf_ref[...] = jnp.where(tie1, b1, -1).astype(jnp.bfloat16)
    q1 = _search_byte(bf_ref, kvec - c_hi)

    s32 = _sortable32(jax.lax.bitcast_convert_type(sims_ref[...] + 0.0, jnp.int32))
    hi = s32 >> 16
    b1 = (s32 >> 8) & 0xFF
    tie2 = (hi == p_hi) & (b1 == q1)
    c_b1 = c_hi + jnp.sum(
        ((hi == p_hi) & (b1 > q1)).astype(jnp.float32), axis=1, keepdims=True
    )
    bf_ref[...] = jnp.where(tie2, s32 & 0xFF, -1).astype(jnp.bfloat16)
    q0 = _search_byte(bf_ref, kvec - c_b1)

    t_bits = (p_hi << 16) | (q1 << 8) | q0
    t_f = jax.lax.bitcast_convert_type(
        jnp.where(t_bits < 0, t_bits ^ _FLIP32, t_bits), jnp.float32
    )

    simc = sims_ref[...] + 0.0
    gt = simc > t_f
    cnt_gt = jnp.sum(gt.astype(jnp.float32), axis=1, keepdims=True)
    sum_gt = jnp.sum(jnp.where(gt, simc, 0.0), axis=1, keepdims=True)
    topk_sum = sum_gt + (np.float32(K) - cnt_gt) * t_f

    idc = idx_ref[0]
    col = jax.lax.broadcasted_iota(jnp.int32, (BR, M), 1)
    pos = jnp.sum(jnp.where(col == idc, simc, 0.0), axis=1, keepdims=True)

    part = jnp.sum(-pos + topk_sum * np.float32(1.0 / K), keepdims=True) * np.float32(
        1.0 / B
    )

    @pl.when(i == 0)
    def _():
        loss_ref[...] = jnp.zeros((1, 1), jnp.float32)

    loss_ref[...] += part


def kernel(points, point_indices, memory_bank):
    mbT = memory_bank.T
    idx3 = point_indices.reshape(NB, BR, 1)

    sims, loss = pl.pallas_call(
        _tc_body,
        grid=(NB,),
        in_specs=[
            pl.BlockSpec((BR, D), lambda i: (i, 0)),
            pl.BlockSpec((1, BR, 1), lambda i: (i, 0, 0)),
            pl.BlockSpec(memory_space=pl.ANY),
        ],
        out_specs=[
            pl.BlockSpec((BR, M), lambda i: (i, 0)),
            pl.BlockSpec((1, 1), lambda i: (0, 0)),
        ],
        out_shape=[
            jax.ShapeDtypeStruct((B, M), jnp.float32),
            jax.ShapeDtypeStruct((1, 1), jnp.float32),
        ],
        scratch_shapes=[
            pltpu.VMEM((BR, M), jnp.bfloat16),
            pltpu.VMEM((D, M), jnp.float32),
            pltpu.SemaphoreType.DMA,
        ],
    )(points, idx3, mbT)

    return loss[0, 0], sims


# speedup vs baseline: 1.8374x; 1.8374x over previous
import jax
import jax.numpy as jnp
import numpy as np
from jax.experimental import pallas as pl
from jax.experimental.pallas import tpu as pltpu

B = 1024
D = 32
M = 100000
K = 4096
BR = 16
NB = B // BR

W = 4096
NFULL = M // W
TAIL = M - NFULL * W

_FLIP32 = np.int32(0x7FFFFFFF)


def _sortable32(bits):
    return jnp.where(bits < 0, bits ^ _FLIP32, bits)


def _count_ge_bf(bf_ref, mid_bf):
    def chunk(c, acc):
        blk = bf_ref[:, pl.ds(c * W, W)]
        return acc + jnp.where(blk >= mid_bf, jnp.bfloat16(1), jnp.bfloat16(0))

    acc = jax.lax.fori_loop(0, NFULL, chunk, jnp.zeros((BR, W), jnp.bfloat16))
    cnt = jnp.sum(acc.astype(jnp.float32), axis=1, keepdims=True)
    tailm = bf_ref[:, pl.ds(NFULL * W, TAIL)] >= mid_bf
    return cnt + jnp.sum(tailm.astype(jnp.float32), axis=1, keepdims=True)


def _search_hi16(bf_ref, k, z):
    lo0 = jnp.full((BR, 1), -32768, jnp.int32)
    hi0 = jnp.full((BR, 1), 32767, jnp.int32)

    def body(_, carry):
        lo, hi = carry
        mid = (lo >> 1) + (hi >> 1) + ((lo | hi) & 1)
        mbits = jnp.where(mid < 0, mid ^ 0x7FFF, mid) << 16
        mid_bf = jax.lax.bitcast_convert_type(mbits, jnp.float32).astype(jnp.bfloat16)
        cnt = _count_ge_bf(bf_ref, mid_bf) - jnp.where(mid == 0, z, 0.0)
        ge = cnt >= k
        return jnp.where(ge, mid, lo), jnp.where(ge, hi, mid - 1)

    lo, _ = jax.lax.fori_loop(0, 16, body, (lo0, hi0))
    return lo


def _search_byte(bf_ref, k):
    lo0 = jnp.full((BR, 1), 0, jnp.int32)
    hi0 = jnp.full((BR, 1), 255, jnp.int32)

    def body(_, carry):
        lo, hi = carry
        mid = (lo + hi + 1) >> 1
        cnt = _count_ge_bf(bf_ref, mid.astype(jnp.float32).astype(jnp.bfloat16))
        ge = cnt >= k
        return jnp.where(ge, mid, lo), jnp.where(ge, hi, mid - 1)

    lo, _ = jax.lax.fori_loop(0, 8, body, (lo0, hi0))
    return lo


def _tc_body(points_ref, idx_ref, mbT_ref, sims_ref, loss_ref, bf_ref, mb_vmem, sem):
    i = pl.program_id(0)

    @pl.when(i == 0)
    def _():
        cp = pltpu.make_async_copy(mbT_ref, mb_vmem, sem)
        cp.start()
        cp.wait()

    p = points_ref[...]
    norm = jnp.sqrt(jnp.sum(p * p, axis=1, keepdims=True))
    pn = p / norm
    sims = jnp.dot(pn, mb_vmem[...], preferred_element_type=jnp.float32)
    sims_ref[...] = sims

    h16 = (jax.lax.bitcast_convert_type(sims + 0.0, jnp.int32) >> 16).astype(
        jnp.int16
    )
    z = jnp.sum((h16 == jnp.int16(-32768)).astype(jnp.float32), axis=1, keepdims=True)
    bf_ref[...] = jax.lax.bitcast_convert_type(h16, jnp.bfloat16)
    kvec = jnp.full((BR, 1), float(K), jnp.float32)
    p_hi = _search_hi16(bf_ref, kvec, z)

    s32 = _sortable32(jax.lax.bitcast_convert_type(sims_ref[...] + 0.0, jnp.int32))
    hi = s32 >> 16
    tie1 = hi == p_hi
    c_hi = jnp.sum((hi > p_hi).astype(jnp.float32), axis=1, keepdims=True)
    b1 = (s32 >> 8) & 0xFF
    bf_ref[...] = jnp.where(tie1, b1.astype(jnp.float32), -1.0).astype(jnp.bfloat16)
    q1 = _search_byte(bf_ref, kvec - c_hi)

    s32 = _sortable32(jax.lax.bitcast_convert_type(sims_ref[...] + 0.0, jnp.int32))
    hi = s32 >> 16
    b1 = (s32 >> 8) & 0xFF
    tie2 = (hi == p_hi) & (b1 == q1)
    c_b1 = c_hi + jnp.sum(
        ((hi == p_hi) & (b1 > q1)).astype(jnp.float32), axis=1, keepdims=True
    )
    bf_ref[...] = jnp.where(
        tie2, (s32 & 0xFF).astype(jnp.float32), -1.0
    ).astype(jnp.bfloat16)
    q0 = _search_byte(bf_ref, kvec - c_b1)

    t_bits = (p_hi << 16) | (q1 << 8) | q0
    t_f = jax.lax.bitcast_convert_type(
        jnp.where(t_bits < 0, t_bits ^ _FLIP32, t_bits), jnp.float32
    )

    simc = sims_ref[...] + 0.0
    gt = simc > t_f
    cnt_gt = jnp.sum(gt.astype(jnp.float32), axis=1, keepdims=True)
    sum_gt = jnp.sum(jnp.where(gt, simc, 0.0), axis=1, keepdims=True)
    topk_sum = sum_gt + (np.float32(K) - cnt_gt) * t_f

    idc = idx_ref[0]
    col = jax.lax.broadcasted_iota(jnp.int32, (BR, M), 1)
    pos = jnp.sum(jnp.where(col == idc, simc, 0.0), axis=1, keepdims=True)

    part = jnp.sum(-pos + topk_sum * np.float32(1.0 / K), keepdims=True) * np.float32(
        1.0 / B
    )

    @pl.when(i == 0)
    def _():
        loss_ref[...] = jnp.zeros((1, 1), jnp.float32)

    loss_ref[...] += part


def kernel(points, point_indices, memory_bank):
    mbT = memory_bank.T
    idx3 = point_indices.reshape(NB, BR, 1)

    sims, loss = pl.pallas_call(
        _tc_body,
        grid=(NB,),
        in_specs=[
            pl.BlockSpec((BR, D), lambda i: (i, 0)),
            pl.BlockSpec((1, BR, 1), lambda i: (i, 0, 0)),
            pl.BlockSpec(memory_space=pl.ANY),
        ],
        out_specs=[
            pl.BlockSpec((BR, M), lambda i: (i, 0)),
            pl.BlockSpec((1, 1), lambda i: (0, 0)),
        ],
        out_shape=[
            jax.ShapeDtypeStruct((B, M), jnp.float32),
            jax.ShapeDtypeStruct((1, 1), jnp.float32),
        ],
        scratch_shapes=[
            pltpu.VMEM((BR, M), jnp.bfloat16),
            pltpu.VMEM((D, M), jnp.float32),
            pltpu.SemaphoreType.DMA,
        ],
    )(points, idx3, mbT)

    return loss[0, 0], sims


# speedup vs baseline: 2.3832x; 1.2971x over previous
import jax
import jax.numpy as jnp
import numpy as np
from jax.experimental import pallas as pl
from jax.experimental.pallas import tpu as pltpu

B = 1024
D = 32
M = 100000
K = 4096
BR = 16
NB = B // BR

W = 4096
NFULL = M // W
TAIL = M - NFULL * W

_FLIP32 = np.int32(0x7FFFFFFF)


def _sortable32(bits):
    return jnp.where(bits < 0, bits ^ _FLIP32, bits)


def _count_ge_bf(bf_ref, mid_bf):
    acc = jnp.zeros((BR, W), jnp.bfloat16)
    for c in range(NFULL):
        blk = bf_ref[:, c * W : (c + 1) * W]
        acc = acc + jnp.where(blk >= mid_bf, jnp.bfloat16(1), jnp.bfloat16(0))
    cnt = jnp.sum(acc.astype(jnp.float32), axis=1, keepdims=True)
    tailm = bf_ref[:, NFULL * W :] >= mid_bf
    return cnt + jnp.sum(tailm.astype(jnp.float32), axis=1, keepdims=True)


def _search_hi16(bf_ref, k, z):
    lo0 = jnp.full((BR, 1), -32768, jnp.int32)
    hi0 = jnp.full((BR, 1), 32767, jnp.int32)

    def body(_, carry):
        lo, hi = carry
        mid = (lo >> 1) + (hi >> 1) + ((lo | hi) & 1)
        mbits = jnp.where(mid < 0, mid ^ 0x7FFF, mid) << 16
        mid_bf = jax.lax.bitcast_convert_type(mbits, jnp.float32).astype(jnp.bfloat16)
        cnt = _count_ge_bf(bf_ref, mid_bf) - jnp.where(mid == 0, z, 0.0)
        ge = cnt >= k
        return jnp.where(ge, mid, lo), jnp.where(ge, hi, mid - 1)

    lo, _ = jax.lax.fori_loop(0, 16, body, (lo0, hi0))
    return lo


def _search_byte(bf_ref, k):
    lo0 = jnp.full((BR, 1), 0, jnp.int32)
    hi0 = jnp.full((BR, 1), 255, jnp.int32)

    def body(_, carry):
        lo, hi = carry
        mid = (lo + hi + 1) >> 1
        cnt = _count_ge_bf(bf_ref, mid.astype(jnp.float32).astype(jnp.bfloat16))
        ge = cnt >= k
        return jnp.where(ge, mid, lo), jnp.where(ge, hi, mid - 1)

    lo, _ = jax.lax.fori_loop(0, 8, body, (lo0, hi0))
    return lo


def _tc_body(points_ref, idx_ref, mbT_ref, sims_ref, loss_ref, bf_ref, mb_vmem, sem):
    i = pl.program_id(0)

    @pl.when(i == 0)
    def _():
        cp = pltpu.make_async_copy(mbT_ref, mb_vmem, sem)
        cp.start()
        cp.wait()

    p = points_ref[...]
    norm = jnp.sqrt(jnp.sum(p * p, axis=1, keepdims=True))
    pn = p / norm
    sims = jnp.dot(pn, mb_vmem[...], preferred_element_type=jnp.float32)
    sims_ref[...] = sims

    h16 = (jax.lax.bitcast_convert_type(sims + 0.0, jnp.int32) >> 16).astype(
        jnp.int16
    )
    z = jnp.sum((h16 == jnp.int16(-32768)).astype(jnp.float32), axis=1, keepdims=True)
    bf_ref[...] = jax.lax.bitcast_convert_type(h16, jnp.bfloat16)
    kvec = jnp.full((BR, 1), float(K), jnp.float32)
    p_hi = _search_hi16(bf_ref, kvec, z)

    s32 = _sortable32(jax.lax.bitcast_convert_type(sims_ref[...] + 0.0, jnp.int32))
    hi = s32 >> 16
    tie1 = hi == p_hi
    c_hi = jnp.sum((hi > p_hi).astype(jnp.float32), axis=1, keepdims=True)
    b1 = (s32 >> 8) & 0xFF
    bf_ref[...] = jnp.where(tie1, b1.astype(jnp.float32), -1.0).astype(jnp.bfloat16)
    q1 = _search_byte(bf_ref, kvec - c_hi)

    s32 = _sortable32(jax.lax.bitcast_convert_type(sims_ref[...] + 0.0, jnp.int32))
    hi = s32 >> 16
    b1 = (s32 >> 8) & 0xFF
    tie2 = (hi == p_hi) & (b1 == q1)
    c_b1 = c_hi + jnp.sum(
        ((hi == p_hi) & (b1 > q1)).astype(jnp.float32), axis=1, keepdims=True
    )
    bf_ref[...] = jnp.where(
        tie2, (s32 & 0xFF).astype(jnp.float32), -1.0
    ).astype(jnp.bfloat16)
    q0 = _search_byte(bf_ref, kvec - c_b1)

    t_bits = (p_hi << 16) | (q1 << 8) | q0
    t_f = jax.lax.bitcast_convert_type(
        jnp.where(t_bits < 0, t_bits ^ _FLIP32, t_bits), jnp.float32
    )

    simc = sims_ref[...] + 0.0
    gt = simc > t_f
    cnt_gt = jnp.sum(gt.astype(jnp.float32), axis=1, keepdims=True)
    sum_gt = jnp.sum(jnp.where(gt, simc, 0.0), axis=1, keepdims=True)
    topk_sum = sum_gt + (np.float32(K) - cnt_gt) * t_f

    idc = idx_ref[0]
    col = jax.lax.broadcasted_iota(jnp.int32, (BR, M), 1)
    pos = jnp.sum(jnp.where(col == idc, simc, 0.0), axis=1, keepdims=True)

    part = jnp.sum(-pos + topk_sum * np.float32(1.0 / K), keepdims=True) * np.float32(
        1.0 / B
    )

    @pl.when(i == 0)
    def _():
        loss_ref[...] = jnp.zeros((1, 1), jnp.float32)

    loss_ref[...] += part


def kernel(points, point_indices, memory_bank):
    mbT = memory_bank.T
    idx3 = point_indices.reshape(NB, BR, 1)

    sims, loss = pl.pallas_call(
        _tc_body,
        grid=(NB,),
        in_specs=[
            pl.BlockSpec((BR, D), lambda i: (i, 0)),
            pl.BlockSpec((1, BR, 1), lambda i: (i, 0, 0)),
            pl.BlockSpec(memory_space=pl.ANY),
        ],
        out_specs=[
            pl.BlockSpec((BR, M), lambda i: (i, 0)),
            pl.BlockSpec((1, 1), lambda i: (0, 0)),
        ],
        out_shape=[
            jax.ShapeDtypeStruct((B, M), jnp.float32),
            jax.ShapeDtypeStruct((1, 1), jnp.float32),
        ],
        scratch_shapes=[
            pltpu.VMEM((BR, M), jnp.bfloat16),
            pltpu.VMEM((D, M), jnp.float32),
            pltpu.SemaphoreType.DMA,
        ],
    )(points, idx3, mbT)

    return loss[0, 0], sims


# speedup vs baseline: 2.3922x; 1.0038x over previous
import functools

import jax
import jax.numpy as jnp
import numpy as np
from jax.experimental import pallas as pl
from jax.experimental.pallas import tpu as pltpu
from jax.experimental.pallas import tpu_sc as plsc

B = 1024
D = 32
M = 100000
K = 4096
BR = 16
NB = B // BR

W = 4096
NFULL = M // W
TAIL = M - NFULL * W

_FLIP32 = np.int32(0x7FFFFFFF)


def _sortable32(bits):
    return jnp.where(bits < 0, bits ^ _FLIP32, bits)


def _count_ge_bf(bf_ref, mid_bf):
    acc = jnp.zeros((BR, W), jnp.bfloat16)
    for c in range(NFULL):
        blk = bf_ref[:, c * W : (c + 1) * W]
        acc = acc + jnp.where(blk >= mid_bf, jnp.bfloat16(1), jnp.bfloat16(0))
    cnt = jnp.sum(acc.astype(jnp.float32), axis=1, keepdims=True)
    tailm = bf_ref[:, NFULL * W :] >= mid_bf
    return cnt + jnp.sum(tailm.astype(jnp.float32), axis=1, keepdims=True)


def _search_hi16(bf_ref, k, z):
    lo0 = jnp.full((BR, 1), -32768, jnp.int32)
    hi0 = jnp.full((BR, 1), 32767, jnp.int32)

    def body(_, carry):
        lo, hi = carry
        mid = (lo >> 1) + (hi >> 1) + ((lo | hi) & 1)
        mbits = jnp.where(mid < 0, mid ^ 0x7FFF, mid) << 16
        mid_bf = jax.lax.bitcast_convert_type(mbits, jnp.float32).astype(jnp.bfloat16)
        cnt = _count_ge_bf(bf_ref, mid_bf) - jnp.where(mid == 0, z, 0.0)
        ge = cnt >= k
        return jnp.where(ge, mid, lo), jnp.where(ge, hi, mid - 1)

    lo, _ = jax.lax.fori_loop(0, 16, body, (lo0, hi0))
    return lo


def _search_byte(bf_ref, k):
    lo0 = jnp.full((BR, 1), 0, jnp.int32)
    hi0 = jnp.full((BR, 1), 255, jnp.int32)

    def body(_, carry):
        lo, hi = carry
        mid = (lo + hi + 1) >> 1
        cnt = _count_ge_bf(bf_ref, mid.astype(jnp.float32).astype(jnp.bfloat16))
        ge = cnt >= k
        return jnp.where(ge, mid, lo), jnp.where(ge, hi, mid - 1)

    lo, _ = jax.lax.fori_loop(0, 8, body, (lo0, hi0))
    return lo


def _tc_body(
    points_ref, mbT_ref, sims_ref, topk_ref, pnorm_ref, bf_ref, mb_vmem, sem
):
    i = pl.program_id(0)

    @pl.when(i == 0)
    def _():
        cp = pltpu.make_async_copy(mbT_ref, mb_vmem, sem)
        cp.start()
        cp.wait()

    p = points_ref[...]
    norm = jnp.sqrt(jnp.sum(p * p, axis=1, keepdims=True))
    pn = p / norm
    pnorm_ref[...] = norm
    sims = jnp.dot(pn, mb_vmem[...], preferred_element_type=jnp.float32)
    sims_ref[...] = sims

    h16 = (jax.lax.bitcast_convert_type(sims + 0.0, jnp.int32) >> 16).astype(
        jnp.int16
    )
    z = jnp.sum((h16 == jnp.int16(-32768)).astype(jnp.float32), axis=1, keepdims=True)
    bf_ref[...] = jax.lax.bitcast_convert_type(h16, jnp.bfloat16)
    kvec = jnp.full((BR, 1), float(K), jnp.float32)
    p_hi = _search_hi16(bf_ref, kvec, z)

    s32 = _sortable32(jax.lax.bitcast_convert_type(sims_ref[...] + 0.0, jnp.int32))
    hi = s32 >> 16
    tie1 = hi == p_hi
    c_hi = jnp.sum((hi > p_hi).astype(jnp.float32), axis=1, keepdims=True)
    b1 = (s32 >> 8) & 0xFF
    bf_ref[...] = jnp.where(tie1, b1.astype(jnp.float32), -1.0).astype(jnp.bfloat16)
    q1 = _search_byte(bf_ref, kvec - c_hi)

    s32 = _sortable32(jax.lax.bitcast_convert_type(sims_ref[...] + 0.0, jnp.int32))
    hi = s32 >> 16
    b1 = (s32 >> 8) & 0xFF
    tie2 = (hi == p_hi) & (b1 == q1)
    c_b1 = c_hi + jnp.sum(
        ((hi == p_hi) & (b1 > q1)).astype(jnp.float32), axis=1, keepdims=True
    )
    bf_ref[...] = jnp.where(
        tie2, (s32 & 0xFF).astype(jnp.float32), -1.0
    ).astype(jnp.bfloat16)
    q0 = _search_byte(bf_ref, kvec - c_b1)

    t_bits = (p_hi << 16) | (q1 << 8) | q0
    t_f = jax.lax.bitcast_convert_type(
        jnp.where(t_bits < 0, t_bits ^ _FLIP32, t_bits), jnp.float32
    )

    simc = sims_ref[...] + 0.0
    gt = simc > t_f
    cnt_gt = jnp.sum(gt.astype(jnp.float32), axis=1, keepdims=True)
    sum_gt = jnp.sum(jnp.where(gt, simc, 0.0), axis=1, keepdims=True)
    topk_ref[...] = sum_gt + (np.float32(K) - cnt_gt) * t_f


_SC_BPW = B // 32


def _sc_body(table_hbm, idx_hbm, pts_hbm, out_hbm, idx_v, rows_v, pts_v, out_v, sem):
    wid = jax.lax.axis_index("s") * 2 + jax.lax.axis_index("c")
    base = wid * _SC_BPW
    pltpu.sync_copy(idx_hbm.at[pl.ds(base, _SC_BPW)], idx_v)
    pltpu.sync_copy(pts_hbm.at[pl.ds(base, _SC_BPW)], pts_v)
    cp = pltpu.make_async_copy(table_hbm.at[idx_v], rows_v, sem)
    cp.start()
    cp.wait()
    lanes = jax.lax.iota(jnp.int32, 16)
    for g in range(_SC_BPW // 16):
        acc = jnp.zeros((16,), jnp.float32)
        for r in range(16):
            row = g * 16 + r
            prod = (
                rows_v[row, pl.ds(0, 16)] * pts_v[row, pl.ds(0, 16)]
                + rows_v[row, pl.ds(16, 16)] * pts_v[row, pl.ds(16, 16)]
            )
            s = prod[0]
            for jj in range(1, 16):
                s = s + prod[jj]
            acc = jnp.where(lanes == r, jnp.full((16,), s, jnp.float32), acc)
        out_v[pl.ds(g * 16, 16)] = acc
    pltpu.sync_copy(out_v, out_hbm.at[pl.ds(base, _SC_BPW)])


def _sc_positive_dots(memory_bank, point_indices, points):
    mesh = plsc.VectorSubcoreMesh(core_axis_name="c", subcore_axis_name="s")
    run = functools.partial(
        pl.kernel,
        mesh=mesh,
        compiler_params=pltpu.CompilerParams(use_tc_tiling_on_sc=False),
        out_type=jax.ShapeDtypeStruct((B,), jnp.float32),
        scratch_types=[
            pltpu.VMEM((_SC_BPW,), jnp.int32),
            pltpu.VMEM((_SC_BPW, D), jnp.float32),
            pltpu.VMEM((_SC_BPW, D), jnp.float32),
            pltpu.VMEM((_SC_BPW,), jnp.float32),
            pltpu.SemaphoreType.DMA,
        ],
    )(_sc_body)
    return run(memory_bank, point_indices, points)


def kernel(points, point_indices, memory_bank):
    mbT = memory_bank.T

    sims, topk, pnorm = pl.pallas_call(
        _tc_body,
        grid=(NB,),
        in_specs=[
            pl.BlockSpec((BR, D), lambda i: (i, 0)),
            pl.BlockSpec(memory_space=pl.ANY),
        ],
        out_specs=[
            pl.BlockSpec((BR, M), lambda i: (i, 0)),
            pl.BlockSpec((BR, 1), lambda i: (i, 0)),
            pl.BlockSpec((BR, 1), lambda i: (i, 0)),
        ],
        out_shape=[
            jax.ShapeDtypeStruct((B, M), jnp.float32),
            jax.ShapeDtypeStruct((B, 1), jnp.float32),
            jax.ShapeDtypeStruct((B, 1), jnp.float32),
        ],
        scratch_shapes=[
            pltpu.VMEM((BR, M), jnp.bfloat16),
            pltpu.VMEM((D, M), jnp.float32),
            pltpu.SemaphoreType.DMA,
        ],
    )(points, mbT)

    raw_pos = _sc_positive_dots(memory_bank, point_indices, points)

    pos = raw_pos / pnorm[:, 0]
    loss = jnp.mean(-pos + topk[:, 0] * np.float32(1.0 / K))
    return loss, sims
